# Initial kernel scaffold; baseline (speedup 1.0000x reference)
import jax
import jax.numpy as jnp
from jax.experimental import pallas as pl


def kernel(points, full_coors, coors_inv, W1, b1, W2, b2):
    raise NotImplementedError("write your pallas kernel here")


# speedup vs baseline: 5.0710x; 5.0710x over previous
import jax
import jax.numpy as jnp
import numpy as np
from jax import lax
from jax.experimental import pallas as pl
from jax.experimental.pallas import tpu as pltpu
from jax.experimental.pallas import tpu_sc as plsc

_N = 640000
_V = 10000
_OUTC = 128
_NC, _NS = 2, 16
_NW = _NC * _NS
_B = 80
_CHUNK = 400
_OPS = _CHUNK // _B
_PPW = _N // _NW
_NCHUNK_W = _PPW // _CHUNK
_CHUNK4 = 160
_OPS4 = _CHUNK4 // _B
_NCHUNK_W4 = _PPW // _CHUNK4
_PPT1 = _N // _NS
_NCHUNK_1 = _PPT1 // _CHUNK
_RPT = 640

_MINS = np.array([-50.0, -50.0, -4.0], np.float32)
_CROP = np.array([100.0, 100.0, 6.0], np.float32)
_SPATIAL = np.array([480.0, 360.0, 32.0], np.float32)
_INTERVALS = _CROP / _SPATIAL


def _tile_voxel_rows(s, fn):
    @pl.when(s < _NS - 1)
    def _():
        fn(pl.multiple_of(s * _RPT, 8), _RPT)

    @pl.when(s == _NS - 1)
    def _():
        fn((_NS - 1) * _RPT, _V - (_NS - 1) * _RPT)


def _k1_body(pg_hbm, idx3_hbm, zeros_hbm, out_hbm, acc, idx2, buf, sem):
    c = lax.axis_index("c")
    s = lax.axis_index("s")

    @pl.when(c == 0)
    def _():
        _tile_voxel_rows(
            s, lambda off, n: pltpu.sync_copy(zeros_hbm.at[pl.ds(off, n)],
                                              acc.at[pl.ds(off, n)]))
    plsc.subcore_barrier()

    @pl.when(c == 0)
    def _():
        pltpu.sync_copy(idx3_hbm.at[s], idx2)

        def body(i, carry):
            roff = s * _PPT1 + i * _CHUNK
            pltpu.sync_copy(pg_hbm.at[pl.ds(roff, _CHUNK)], buf)
            descs = [
                pltpu.async_copy(buf.at[pl.ds(j * _B, _B)],
                                 acc.at[idx2.at[i * _OPS + j]],
                                 sem, add=True)
                for j in range(_OPS)
            ]
            for d in descs:
                d.wait()
            return carry

        lax.fori_loop(0, _NCHUNK_1, body, 0)
    plsc.subcore_barrier()

    @pl.when(c == 0)
    def _():
        _tile_voxel_rows(
            s, lambda off, n: pltpu.sync_copy(acc.at[pl.ds(off, n)],
                                              out_hbm.at[pl.ds(off, n)]))


def _k2_body(tab_hbm, idx3_hbm, out_hbm, idx2, buf, sem):
    c = lax.axis_index("c")
    s = lax.axis_index("s")
    w = s * _NC + c
    pltpu.sync_copy(idx3_hbm.at[w], idx2)

    def body(i, carry):
        roff = w * _PPW + i * _CHUNK
        descs = [
            pltpu.async_copy(tab_hbm.at[idx2.at[i * _OPS + j]],
                             buf.at[pl.ds(j * _B, _B)], sem)
            for j in range(_OPS)
        ]
        for d in descs:
            d.wait()
        pltpu.sync_copy(buf, out_hbm.at[pl.ds(roff, _CHUNK)])
        return carry

    lax.fori_loop(0, _NCHUNK_W, body, 0)


def _k4_body(h_hbm, idx3_hbm, zeros_hbm, out_hbm, acc, idx2, buf, sem):
    c = lax.axis_index("c")
    s = lax.axis_index("s")
    w = s * _NC + c
    _tile_voxel_rows(
        s, lambda off, n: pltpu.sync_copy(zeros_hbm.at[pl.ds(off, n)],
                                          acc.at[pl.ds(off, n)]))
    plsc.subcore_barrier()
    pltpu.sync_copy(idx3_hbm.at[w], idx2)

    def body(i, carry):
        roff = w * _PPW + i * _CHUNK4
        pltpu.sync_copy(h_hbm.at[pl.ds(roff, _CHUNK4)], buf)
        descs = [
            pltpu.async_copy(buf.at[pl.ds(j * _B, _B)],
                             acc.at[idx2.at[i * _OPS4 + j]],
                             sem, add=True)
            for j in range(_OPS4)
        ]
        for d in descs:
            d.wait()
        return carry

    lax.fori_loop(0, _NCHUNK_W4, body, 0)
    plsc.subcore_barrier()
    cbase = pl.multiple_of(c * _V, 8)
    _tile_voxel_rows(
        s, lambda off, n: pltpu.sync_copy(acc.at[pl.ds(off, n)],
                                          out_hbm.at[pl.ds(cbase + off, n)]))


_BN = 3200
_BV = 2000


def _k3_body(pg_ref, sg_ref, ab_ref, c8_ref, c0_ref, h_ref):
    sg = sg_ref[...]
    cnt = jnp.maximum(sg[:, 7:8], 1.0)
    pre = (jnp.dot(pg_ref[...], ab_ref[...], preferred_element_type=jnp.float32)
           + jnp.dot(sg / cnt, c8_ref[...], preferred_element_type=jnp.float32)
           + c0_ref[...])
    h_ref[...] = jnp.maximum(pre, 0.0)


def _k5_body(p_ref, sg_ref, w2_ref, b2_ref, out_ref):
    p = p_ref[0] + p_ref[1]
    cnt = sg_ref[...][:, 7:8]
    pm = p * (1.0 / jnp.maximum(cnt, 1.0))
    o = jnp.dot(pm, w2_ref[...], preferred_element_type=jnp.float32) + b2_ref[...]
    out_ref[...] = jnp.where(cnt > 0.0, o, 0.0)


def kernel(points, full_coors, coors_inv, W1, b1, W2, b2):
    f32 = jnp.float32
    idx = coors_inv.astype(jnp.int32)
    idx3_k1 = idx.reshape(_NS, _PPT1 // _B, _B)
    idx3_w = idx.reshape(_NW, _PPW // _B, _B)
    gridf = full_coors[:, 1:4].astype(f32)
    pg = jnp.concatenate([points, gridf, jnp.ones((_N, 1), f32)], axis=1)

    W1 = W1.astype(f32)
    A = W1[:4] + jnp.concatenate(
        [W1[4:7] + W1[7:10], jnp.zeros((1, _OUTC), f32)], axis=0)
    B3 = -jnp.asarray(_INTERVALS)[:, None] * W1[7:10]
    AB = jnp.concatenate([A, B3, jnp.zeros((1, _OUTC), f32)], axis=0)
    C8 = jnp.concatenate([-W1[4:7], jnp.zeros((5, _OUTC), f32)], axis=0)
    c0 = (b1 - jnp.asarray(_MINS) @ W1[7:10]).reshape(1, _OUTC)

    zeros8 = jnp.zeros((_V, 8), f32)
    zeros128 = jnp.zeros((_V, _OUTC), f32)
    mesh = plsc.VectorSubcoreMesh(core_axis_name="c", subcore_axis_name="s")
    sc_params = pltpu.CompilerParams(use_tc_tiling_on_sc=False)

    k1 = pl.kernel(
        _k1_body,
        out_type=jax.ShapeDtypeStruct((_V, 8), f32),
        mesh=mesh,
        compiler_params=sc_params,
        scratch_types=[
            pltpu.VMEM_SHARED((_V, 8), f32),
            pltpu.VMEM((_PPT1 // _B, _B), jnp.int32),
            pltpu.VMEM((_CHUNK, 8), f32),
            pltpu.SemaphoreType.DMA,
        ],
    )
    sums8 = k1(pg, idx3_k1, zeros8)

    k2 = pl.kernel(
        _k2_body,
        out_type=jax.ShapeDtypeStruct((_N, 8), f32),
        mesh=mesh,
        compiler_params=sc_params,
        scratch_types=[
            pltpu.VMEM((_PPW // _B, _B), jnp.int32),
            pltpu.VMEM((_CHUNK, 8), f32),
            pltpu.SemaphoreType.DMA,
        ],
    )
    sg = k2(sums8, idx3_w)

    h = pl.pallas_call(
        _k3_body,
        grid=(_N // _BN,),
        in_specs=[
            pl.BlockSpec((_BN, 8), lambda i: (i, 0)),
            pl.BlockSpec((_BN, 8), lambda i: (i, 0)),
            pl.BlockSpec((8, _OUTC), lambda i: (0, 0)),
            pl.BlockSpec((8, _OUTC), lambda i: (0, 0)),
            pl.BlockSpec((1, _OUTC), lambda i: (0, 0)),
        ],
        out_specs=pl.BlockSpec((_BN, _OUTC), lambda i: (i, 0)),
        out_shape=jax.ShapeDtypeStruct((_N, _OUTC), f32),
    )(pg, sg, AB, C8, c0)

    k4 = pl.kernel(
        _k4_body,
        out_type=jax.ShapeDtypeStruct((2 * _V, _OUTC), f32),
        mesh=mesh,
        compiler_params=sc_params,
        scratch_types=[
            pltpu.VMEM_SHARED((_V, _OUTC), f32),
            pltpu.VMEM((_PPW // _B, _B), jnp.int32),
            pltpu.VMEM((_CHUNK4, _OUTC), f32),
            pltpu.SemaphoreType.DMA,
        ],
    )
    partials = k4(h, idx3_w, zeros128).reshape(2, _V, _OUTC)

    out = pl.pallas_call(
        _k5_body,
        grid=(_V // _BV,),
        in_specs=[
            pl.BlockSpec((2, _BV, _OUTC), lambda i: (0, i, 0)),
            pl.BlockSpec((_BV, 8), lambda i: (i, 0)),
            pl.BlockSpec((_OUTC, _OUTC), lambda i: (0, 0)),
            pl.BlockSpec((1, _OUTC), lambda i: (0, 0)),
        ],
        out_specs=pl.BlockSpec((_BV, _OUTC), lambda i: (i, 0)),
        out_shape=jax.ShapeDtypeStruct((_V, _OUTC), f32),
    )(partials, sums8, W2, b2.reshape(1, _OUTC))
    return out


# speedup vs baseline: 9.2892x; 1.8318x over previous
import jax
import jax.numpy as jnp
import numpy as np
from jax import lax
from jax.experimental import pallas as pl
from jax.experimental.pallas import tpu as pltpu
from jax.experimental.pallas import tpu_sc as plsc

_N = 640000
_V = 10000
_OUTC = 128
_NC, _NS = 2, 16
_NW = _NC * _NS
_B = 80
_CHUNK = 400
_OPS = _CHUNK // _B
_PPW = _N // _NW
_NCHUNK_W = _PPW // _CHUNK
_CHUNK4 = 160
_OPS4 = _CHUNK4 // _B
_NCHUNK_W4 = _PPW // _CHUNK4
_PPT1 = _N // _NS
_NCHUNK_1 = _PPT1 // _CHUNK
_RPT = 640

_MINS = np.array([-50.0, -50.0, -4.0], np.float32)
_CROP = np.array([100.0, 100.0, 6.0], np.float32)
_SPATIAL = np.array([480.0, 360.0, 32.0], np.float32)
_INTERVALS = _CROP / _SPATIAL


def _tile_voxel_rows(s, fn):
    @pl.when(s < _NS - 1)
    def _():
        fn(pl.multiple_of(s * _RPT, 8), _RPT)

    @pl.when(s == _NS - 1)
    def _():
        fn((_NS - 1) * _RPT, _V - (_NS - 1) * _RPT)


def _k1_body(pts_hbm, idx3_hbm, zeros_hbm, out_hbm, acc, idx2, buf, cbuf, sem):
    c = lax.axis_index("c")
    s = lax.axis_index("s")

    @pl.when(c == 0)
    def _():
        _tile_voxel_rows(
            s, lambda off, n: pltpu.sync_copy(zeros_hbm.at[pl.ds(off, n)],
                                              acc.at[pl.ds(off, n)]))
    plsc.subcore_barrier()

    @pl.when(c == 0)
    def _():
        pltpu.sync_copy(idx3_hbm.at[s], idx2)
        lanes = lax.iota(jnp.int32, 16)
        ones7 = jnp.where(lanes % 8 == 7, 1.0, 0.0)

        def initb(k, carry):
            plsc.store_scatter(buf, [lanes // 8 + 2 * k, lanes % 8], ones7)
            return carry
        lax.fori_loop(0, _CHUNK // 2, initb, 0)

        def body(i, carry):
            roff = s * _PPT1 + i * _CHUNK
            pltpu.sync_copy(pts_hbm.at[pl.ds(0, 3), pl.ds(roff, _CHUNK)], cbuf)
            for g in range(_CHUNK // 16):
                rows = lanes + g * 16
                for ch in range(3):
                    v = cbuf[ch, pl.ds(g * 16, 16)]
                    plsc.store_scatter(buf, [rows, jnp.full((16,), ch, jnp.int32)], v)
            descs = [
                pltpu.async_copy(buf.at[pl.ds(j * _B, _B)],
                                 acc.at[idx2.at[i * _OPS + j]],
                                 sem, add=True)
                for j in range(_OPS)
            ]
            for d in descs:
                d.wait()
            return carry

        lax.fori_loop(0, _NCHUNK_1, body, 0)
    plsc.subcore_barrier()

    @pl.when(c == 0)
    def _():
        _tile_voxel_rows(
            s, lambda off, n: pltpu.sync_copy(acc.at[pl.ds(off, n)],
                                              out_hbm.at[pl.ds(off, n)]))


def _k2_body(tab_hbm, idx3_hbm, out_hbm, idx2, buf, sem):
    c = lax.axis_index("c")
    s = lax.axis_index("s")
    w = s * _NC + c
    pltpu.sync_copy(idx3_hbm.at[w], idx2)

    def body(i, carry):
        roff = w * _PPW + i * _CHUNK
        descs = [
            pltpu.async_copy(tab_hbm.at[idx2.at[i * _OPS + j]],
                             buf.at[pl.ds(j * _B, _B)], sem)
            for j in range(_OPS)
        ]
        for d in descs:
            d.wait()
        pltpu.sync_copy(buf, out_hbm.at[pl.ds(roff, _CHUNK)])
        return carry

    lax.fori_loop(0, _NCHUNK_W, body, 0)


def _k4_body(h_hbm, idx3_hbm, zeros_hbm, out_hbm, acc, idx2, buf, sem):
    c = lax.axis_index("c")
    s = lax.axis_index("s")
    w = s * _NC + c
    _tile_voxel_rows(
        s, lambda off, n: pltpu.sync_copy(zeros_hbm.at[pl.ds(off, n)],
                                          acc.at[pl.ds(off, n)]))
    plsc.subcore_barrier()
    pltpu.sync_copy(idx3_hbm.at[w], idx2)

    def body(i, carry):
        roff = w * _PPW + i * _CHUNK4
        pltpu.sync_copy(h_hbm.at[pl.ds(roff, _CHUNK4)], buf)
        descs = [
            pltpu.async_copy(buf.at[pl.ds(j * _B, _B)],
                             acc.at[idx2.at[i * _OPS4 + j]],
                             sem, add=True)
            for j in range(_OPS4)
        ]
        for d in descs:
            d.wait()
        return carry

    lax.fori_loop(0, _NCHUNK_W4, body, 0)
    plsc.subcore_barrier()
    cbase = pl.multiple_of(c * _V, 8)
    _tile_voxel_rows(
        s, lambda off, n: pltpu.sync_copy(acc.at[pl.ds(off, n)],
                                          out_hbm.at[pl.ds(cbase + off, n)]))


_BN = 3200
_BV = _V


def _k3_body(pts_ref, grid_ref, sg_ref, a_ref, b_ref, c816_ref, s_ref,
             c0_ref, h_ref):
    f32 = jnp.float32
    sg16 = sg_ref[...]
    cntb = jnp.dot(sg16, s_ref[...], preferred_element_type=f32)
    sgd = sg16 * (1.0 / jnp.maximum(cntb, 1.0))
    mterm = jnp.dot(sgd, c816_ref[...],
                    preferred_element_type=f32).reshape(_BN, _OUTC)
    cdims = (((0,), (0,)), ((), ()))
    pterm = (lax.dot_general(pts_ref[...], a_ref[...], cdims,
                             preferred_element_type=f32)
             + lax.dot_general(grid_ref[...], b_ref[...], cdims,
                               preferred_element_type=f32))
    h_ref[...] = jnp.maximum(pterm + mterm + c0_ref[...], 0.0)


def _k5_body(p_ref, sg_ref, scol_ref, w2_ref, b2_ref, out_ref):
    f32 = jnp.float32
    p = p_ref[0] + p_ref[1]
    cnt16 = jnp.dot(sg_ref[...], scol_ref[...],
                    preferred_element_type=f32)
    cnt = cnt16.reshape(_BV, _OUTC)
    pm = p * (1.0 / jnp.maximum(cnt, 1.0))
    o = jnp.dot(pm, w2_ref[...], preferred_element_type=f32) + b2_ref[...]
    out_ref[...] = jnp.where(cnt > 0.0, o, 0.0)


def kernel(points, full_coors, coors_inv, W1, b1, W2, b2):
    f32 = jnp.float32
    idx = coors_inv.astype(jnp.int32)
    idx3_k1 = idx.reshape(_NS, _PPT1 // _B, _B)
    idx3_w = idx.reshape(_NW, _PPW // _B, _B)
    gridf = full_coors[:, 1:4].astype(f32)

    W1 = W1.astype(f32)
    A = W1[:4] + jnp.concatenate(
        [W1[4:7] + W1[7:10], jnp.zeros((1, _OUTC), f32)], axis=0)
    B3 = -jnp.asarray(_INTERVALS)[:, None] * W1[7:10]
    C8 = jnp.concatenate([-W1[4:7], jnp.zeros((5, _OUTC), f32)], axis=0)
    C816 = jnp.kron(jnp.eye(16, dtype=f32), C8)
    lane = jnp.arange(128)
    S = (((lane[:, None] // 8) == (lane[None, :] // 8))
         & (lane[:, None] % 8 == 7)).astype(f32)
    e7 = (jnp.arange(8)[:, None] == 7).astype(f32) * jnp.ones((1, _OUTC), f32)
    Scol = jnp.kron(jnp.eye(16, dtype=f32), e7)
    c0 = (b1 - jnp.asarray(_MINS) @ W1[7:10]).reshape(1, _OUTC)
    pts_t = points.T
    grid_t = gridf.T

    zeros8 = jnp.zeros((_V, 8), f32)
    zeros128 = jnp.zeros((_V, _OUTC), f32)
    mesh = plsc.VectorSubcoreMesh(core_axis_name="c", subcore_axis_name="s")
    sc_params = pltpu.CompilerParams(use_tc_tiling_on_sc=False,
                                    needs_layout_passes=False)

    k1 = pl.kernel(
        _k1_body,
        out_type=jax.ShapeDtypeStruct((_V, 8), f32),
        mesh=mesh,
        compiler_params=sc_params,
        scratch_types=[
            pltpu.VMEM_SHARED((_V, 8), f32),
            pltpu.VMEM((_PPT1 // _B, _B), jnp.int32),
            pltpu.VMEM((_CHUNK, 8), f32),
            pltpu.VMEM((3, _CHUNK), f32),
            pltpu.SemaphoreType.DMA,
        ],
    )
    sums8 = k1(pts_t, idx3_k1, zeros8)

    k2 = pl.kernel(
        _k2_body,
        out_type=jax.ShapeDtypeStruct((_N, 8), f32),
        mesh=mesh,
        compiler_params=sc_params,
        scratch_types=[
            pltpu.VMEM((_PPW // _B, _B), jnp.int32),
            pltpu.VMEM((_CHUNK, 8), f32),
            pltpu.SemaphoreType.DMA,
        ],
    )
    sg = k2(sums8, idx3_w)

    h = pl.pallas_call(
        _k3_body,
        grid=(_N // _BN,),
        in_specs=[
            pl.BlockSpec((4, _BN), lambda i: (0, i)),
            pl.BlockSpec((3, _BN), lambda i: (0, i)),
            pl.BlockSpec((_BN // 16, 128), lambda i: (i, 0)),
            pl.BlockSpec((4, _OUTC), lambda i: (0, 0)),
            pl.BlockSpec((3, _OUTC), lambda i: (0, 0)),
            pl.BlockSpec((128, 16 * _OUTC), lambda i: (0, 0)),
            pl.BlockSpec((128, 128), lambda i: (0, 0)),
            pl.BlockSpec((1, _OUTC), lambda i: (0, 0)),
        ],
        out_specs=pl.BlockSpec((_BN, _OUTC), lambda i: (i, 0)),
        out_shape=jax.ShapeDtypeStruct((_N, _OUTC), f32),
    )(pts_t, grid_t, sg.reshape(_N // 16, 128), A, B3, C816, S, c0)

    k4 = pl.kernel(
        _k4_body,
        out_type=jax.ShapeDtypeStruct((2 * _V, _OUTC), f32),
        mesh=mesh,
        compiler_params=sc_params,
        scratch_types=[
            pltpu.VMEM_SHARED((_V, _OUTC), f32),
            pltpu.VMEM((_PPW // _B, _B), jnp.int32),
            pltpu.VMEM((_CHUNK4, _OUTC), f32),
            pltpu.SemaphoreType.DMA,
        ],
    )
    partials = k4(h, idx3_w, zeros128).reshape(2, _V, _OUTC)

    out = pl.pallas_call(
        _k5_body,
        grid=(1,),
        in_specs=[
            pl.BlockSpec((2, _BV, _OUTC), lambda i: (0, 0, 0)),
            pl.BlockSpec((_BV // 16, 128), lambda i: (0, 0)),
            pl.BlockSpec((128, 16 * _OUTC), lambda i: (0, 0)),
            pl.BlockSpec((_OUTC, _OUTC), lambda i: (0, 0)),
            pl.BlockSpec((1, _OUTC), lambda i: (0, 0)),
        ],
        out_specs=pl.BlockSpec((_BV, _OUTC), lambda i: (0, 0)),
        out_shape=jax.ShapeDtypeStruct((_V, _OUTC), f32),
    )(partials, sums8.reshape(_V // 16, 128), Scol, W2, b2.reshape(1, _OUTC))
    return out


# speedup vs baseline: 10.5131x; 1.1318x over previous
import jax
import jax.numpy as jnp
import numpy as np
from jax import lax
from jax.experimental import pallas as pl
from jax.experimental.pallas import tpu as pltpu
from jax.experimental.pallas import tpu_sc as plsc

_N = 640000
_V = 10000
_OUTC = 128
_NC, _NS = 2, 16
_NW = _NC * _NS
_B = 80
_CHUNK = 400
_OPS = _CHUNK // _B
_PPW = _N // _NW
_NCHUNK_W = _PPW // _CHUNK
_CHUNK4 = 80
_NCHUNK_W4 = _PPW // _CHUNK4
_PPT1 = _N // _NS
_NCHUNK_1 = _PPT1 // _CHUNK
_RPT = 640

_MINS = np.array([-50.0, -50.0, -4.0], np.float32)
_CROP = np.array([100.0, 100.0, 6.0], np.float32)
_SPATIAL = np.array([480.0, 360.0, 32.0], np.float32)
_INTERVALS = _CROP / _SPATIAL


def _tile_voxel_rows(s, fn):
    @pl.when(s < _NS - 1)
    def _():
        fn(pl.multiple_of(s * _RPT, 8), _RPT)

    @pl.when(s == _NS - 1)
    def _():
        fn((_NS - 1) * _RPT, _V - (_NS - 1) * _RPT)


def _k1_body(pts_hbm, idx3_hbm, zeros_hbm, out_hbm, acc, idx2, buf, cbuf, sem):
    c = lax.axis_index("c")
    s = lax.axis_index("s")

    @pl.when(c == 0)
    def _():
        _tile_voxel_rows(
            s, lambda off, n: pltpu.sync_copy(zeros_hbm.at[pl.ds(off, n)],
                                              acc.at[pl.ds(off, n)]))
    plsc.subcore_barrier()

    @pl.when(c == 0)
    def _():
        pltpu.sync_copy(idx3_hbm.at[s], idx2)
        lanes = lax.iota(jnp.int32, 16)
        ones7 = jnp.where(lanes % 8 == 7, 1.0, 0.0)

        def initb(k, carry):
            plsc.store_scatter(buf, [lanes // 8 + 2 * k, lanes % 8], ones7)
            return carry
        lax.fori_loop(0, _CHUNK // 2, initb, 0)

        def body(i, carry):
            roff = s * _PPT1 + i * _CHUNK
            pltpu.sync_copy(pts_hbm.at[pl.ds(0, 3), pl.ds(roff, _CHUNK)], cbuf)
            for g in range(_CHUNK // 16):
                rows = lanes + g * 16
                for ch in range(3):
                    v = cbuf[ch, pl.ds(g * 16, 16)]
                    plsc.store_scatter(buf, [rows, jnp.full((16,), ch, jnp.int32)], v)
            descs = [
                pltpu.async_copy(buf.at[pl.ds(j * _B, _B)],
                                 acc.at[idx2.at[i * _OPS + j]],
                                 sem, add=True)
                for j in range(_OPS)
            ]
            for d in descs:
                d.wait()
            return carry

        lax.fori_loop(0, _NCHUNK_1, body, 0)
    plsc.subcore_barrier()

    @pl.when(c == 0)
    def _():
        _tile_voxel_rows(
            s, lambda off, n: pltpu.sync_copy(acc.at[pl.ds(off, n)],
                                              out_hbm.at[pl.ds(off, n)]))


def _k2_body(tab_hbm, idx3_hbm, out_hbm, idx2, buf, sem):
    c = lax.axis_index("c")
    s = lax.axis_index("s")
    w = s * _NC + c
    pltpu.sync_copy(idx3_hbm.at[w], idx2)

    def body(i, carry):
        roff = w * _PPW + i * _CHUNK
        descs = [
            pltpu.async_copy(tab_hbm.at[idx2.at[i * _OPS + j]],
                             buf.at[pl.ds(j * _B, _B)], sem)
            for j in range(_OPS)
        ]
        for d in descs:
            d.wait()
        pltpu.sync_copy(buf, out_hbm.at[pl.ds(roff, _CHUNK)])
        return carry

    lax.fori_loop(0, _NCHUNK_W, body, 0)


def _k4_body(h_hbm, idx3_hbm, zeros_hbm, out_hbm, acc, idx2, bufa, bufb,
             sema, semb, sems):
    c = lax.axis_index("c")
    s = lax.axis_index("s")
    w = s * _NC + c
    _tile_voxel_rows(
        s, lambda off, n: pltpu.sync_copy(zeros_hbm.at[pl.ds(off, n)],
                                          acc.at[pl.ds(off, n)]))
    plsc.subcore_barrier()
    pltpu.sync_copy(idx3_hbm.at[w], idx2)
    base = w * _PPW

    def rows(ch):
        return h_hbm.at[pl.ds(base + ch * _CHUNK4, _CHUNK4)]

    pltpu.async_copy(rows(0), bufa, sema)
    npair = _NCHUNK_W4 // 2

    def body(i, carry):
        pltpu.async_copy(rows(2 * i + 1), bufb, semb)
        pltpu.make_async_copy(rows(2 * i), bufa, sema).wait()
        pltpu.async_copy(bufa, acc.at[idx2.at[2 * i]], sems, add=True).wait()

        @pl.when(i < npair - 1)
        def _():
            pltpu.async_copy(rows(2 * i + 2), bufa, sema)
        pltpu.make_async_copy(rows(2 * i + 1), bufb, semb).wait()
        pltpu.async_copy(bufb, acc.at[idx2.at[2 * i + 1]], sems, add=True).wait()
        return carry

    lax.fori_loop(0, npair, body, 0)
    plsc.subcore_barrier()
    cbase = pl.multiple_of(c * _V, 8)
    _tile_voxel_rows(
        s, lambda off, n: pltpu.sync_copy(acc.at[pl.ds(off, n)],
                                          out_hbm.at[pl.ds(cbase + off, n)]))


_BN = 3200
_BV = _V


def _k3_body(pts_ref, grid_ref, sg_ref, a_ref, b_ref, c816_ref, s_ref,
             c0_ref, h_ref):
    f32 = jnp.float32
    sg16 = sg_ref[...]
    cntb = jnp.dot(sg16, s_ref[...], preferred_element_type=f32)
    sgd = sg16 * (1.0 / jnp.maximum(cntb, 1.0))
    mterm = jnp.dot(sgd, c816_ref[...],
                    preferred_element_type=f32).reshape(_BN, _OUTC)
    cdims = (((0,), (0,)), ((), ()))
    pterm = (lax.dot_general(pts_ref[...], a_ref[...], cdims,
                             preferred_element_type=f32)
             + lax.dot_general(grid_ref[...], b_ref[...], cdims,
                               preferred_element_type=f32))
    h_ref[...] = jnp.maximum(pterm + mterm + c0_ref[...], 0.0)


def _k5_body(p_ref, sg_ref, scol_ref, w2_ref, b2_ref, out_ref):
    f32 = jnp.float32
    p = p_ref[0] + p_ref[1]
    cnt16 = jnp.dot(sg_ref[...], scol_ref[...],
                    preferred_element_type=f32)
    cnt = cnt16.reshape(_BV, _OUTC)
    pm = p * (1.0 / jnp.maximum(cnt, 1.0))
    o = jnp.dot(pm, w2_ref[...], preferred_element_type=f32) + b2_ref[...]
    out_ref[...] = jnp.where(cnt > 0.0, o, 0.0)


def kernel(points, full_coors, coors_inv, W1, b1, W2, b2):
    f32 = jnp.float32
    idx = coors_inv.astype(jnp.int32)
    idx3_k1 = idx.reshape(_NS, _PPT1 // _B, _B)
    idx3_w = idx.reshape(_NW, _PPW // _B, _B)
    idx4_w = idx.reshape(_NW, _PPW // _CHUNK4, _CHUNK4)
    gridf = full_coors[:, 1:4].astype(f32)

    W1 = W1.astype(f32)
    A = W1[:4] + jnp.concatenate(
        [W1[4:7] + W1[7:10], jnp.zeros((1, _OUTC), f32)], axis=0)
    B3 = -jnp.asarray(_INTERVALS)[:, None] * W1[7:10]
    C8 = jnp.concatenate([-W1[4:7], jnp.zeros((5, _OUTC), f32)], axis=0)
    C816 = jnp.kron(jnp.eye(16, dtype=f32), C8)
    lane = jnp.arange(128)
    S = (((lane[:, None] // 8) == (lane[None, :] // 8))
         & (lane[:, None] % 8 == 7)).astype(f32)
    e7 = (jnp.arange(8)[:, None] == 7).astype(f32) * jnp.ones((1, _OUTC), f32)
    Scol = jnp.kron(jnp.eye(16, dtype=f32), e7)
    c0 = (b1 - jnp.asarray(_MINS) @ W1[7:10]).reshape(1, _OUTC)
    pts_t = points.T
    grid_t = gridf.T

    zeros8 = jnp.zeros((_V, 8), f32)
    zeros128 = jnp.zeros((_V, _OUTC), f32)
    mesh = plsc.VectorSubcoreMesh(core_axis_name="c", subcore_axis_name="s")
    sc_params = pltpu.CompilerParams(use_tc_tiling_on_sc=False,
                                    needs_layout_passes=False)

    k1 = pl.kernel(
        _k1_body,
        out_type=jax.ShapeDtypeStruct((_V, 8), f32),
        mesh=mesh,
        compiler_params=sc_params,
        scratch_types=[
            pltpu.VMEM_SHARED((_V, 8), f32),
            pltpu.VMEM((_PPT1 // _B, _B), jnp.int32),
            pltpu.VMEM((_CHUNK, 8), f32),
            pltpu.VMEM((3, _CHUNK), f32),
            pltpu.SemaphoreType.DMA,
        ],
    )
    sums8 = k1(pts_t, idx3_k1, zeros8)

    k2 = pl.kernel(
        _k2_body,
        out_type=jax.ShapeDtypeStruct((_N, 8), f32),
        mesh=mesh,
        compiler_params=sc_params,
        scratch_types=[
            pltpu.VMEM((_PPW // _B, _B), jnp.int32),
            pltpu.VMEM((_CHUNK, 8), f32),
            pltpu.SemaphoreType.DMA,
        ],
    )
    sg = k2(sums8, idx3_w)

    h = pl.pallas_call(
        _k3_body,
        grid=(_N // _BN,),
        in_specs=[
            pl.BlockSpec((4, _BN), lambda i: (0, i)),
            pl.BlockSpec((3, _BN), lambda i: (0, i)),
            pl.BlockSpec((_BN // 16, 128), lambda i: (i, 0)),
            pl.BlockSpec((4, _OUTC), lambda i: (0, 0)),
            pl.BlockSpec((3, _OUTC), lambda i: (0, 0)),
            pl.BlockSpec((128, 16 * _OUTC), lambda i: (0, 0)),
            pl.BlockSpec((128, 128), lambda i: (0, 0)),
            pl.BlockSpec((1, _OUTC), lambda i: (0, 0)),
        ],
        out_specs=pl.BlockSpec((_BN, _OUTC), lambda i: (i, 0)),
        out_shape=jax.ShapeDtypeStruct((_N, _OUTC), f32),
    )(pts_t, grid_t, sg.reshape(_N // 16, 128), A, B3, C816, S, c0)

    k4 = pl.kernel(
        _k4_body,
        out_type=jax.ShapeDtypeStruct((2 * _V, _OUTC), f32),
        mesh=mesh,
        compiler_params=sc_params,
        scratch_types=[
            pltpu.VMEM_SHARED((_V, _OUTC), f32),
            pltpu.VMEM((_PPW // _CHUNK4, _CHUNK4), jnp.int32),
            pltpu.VMEM((_CHUNK4, _OUTC), f32),
            pltpu.VMEM((_CHUNK4, _OUTC), f32),
            pltpu.SemaphoreType.DMA,
            pltpu.SemaphoreType.DMA,
            pltpu.SemaphoreType.DMA,
        ],
    )
    partials = k4(h, idx4_w, zeros128).reshape(2, _V, _OUTC)

    out = pl.pallas_call(
        _k5_body,
        grid=(1,),
        in_specs=[
            pl.BlockSpec((2, _BV, _OUTC), lambda i: (0, 0, 0)),
            pl.BlockSpec((_BV // 16, 128), lambda i: (0, 0)),
            pl.BlockSpec((128, 16 * _OUTC), lambda i: (0, 0)),
            pl.BlockSpec((_OUTC, _OUTC), lambda i: (0, 0)),
            pl.BlockSpec((1, _OUTC), lambda i: (0, 0)),
        ],
        out_specs=pl.BlockSpec((_BV, _OUTC), lambda i: (0, 0)),
        out_shape=jax.ShapeDtypeStruct((_V, _OUTC), f32),
    )(partials, sums8.reshape(_V // 16, 128), Scol, W2, b2.reshape(1, _OUTC))
    return out
